# baseline (speedup 1.0000x reference)
import jax
import jax.numpy as jnp
from jax import lax
from jax.experimental import pallas as pl
from jax.experimental.pallas import tpu as pltpu
from jax.experimental.pallas import tpu_sc as plsc

BATCH = 16384
DIM = 64
HALF = 32
MARGIN = 1.0

_NC = 2
_NS = 16
_NW = _NC * _NS
_BPW = BATCH // _NW
_GCH = 128
_NG = _BPW // _GCH

_NU = 1000000
_TW = 512
_STRIPE = 31232
_NPAIR = 30
_XBASE = _NW * _STRIPE
_AUX0 = _XBASE + 4 * 128
_TRASH = BATCH
_HGR = BATCH + 128


def _tc_prep_body(rel_ref, tail_ref, cos_ref, sin_ref, aux_ref):
    r = rel_ref[...]
    cos_ref[...] = jnp.cos(r)
    sin_ref[...] = jnp.sin(r)
    aux_ref[...] = tail_ref[...].T


_tc_prep = pl.pallas_call(
    _tc_prep_body,
    out_shape=[
        jax.ShapeDtypeStruct((3, HALF), jnp.float32),
        jax.ShapeDtypeStruct((3, HALF), jnp.float32),
        jax.ShapeDtypeStruct((64, 64), jnp.float32),
    ],
)


def _ga_body(vtu, heads, hg,
             heads_v, wl_p, buf0, buf1, stage, pos_v,
             tmp_h, tmp_p, ns_ref, sem, sem2):
    wid = lax.axis_index("s") * _NC + lax.axis_index("c")
    lo = wid * _STRIPE
    hi = lo + _STRIPE
    xlo = _XBASE + wid * 128
    xhi = xlo + 128
    lanes = lax.iota(jnp.int32, 16)

    pltpu.sync_copy(heads, heads_v.at[pl.ds(0, BATCH)])

    for s in range(8):
        pos_v[pl.ds(s * 16, 16)] = jnp.full((16,), _TRASH, jnp.int32)

    heads_v[pl.ds(BATCH, 16)] = jnp.full((16,), -1, jnp.int32)

    def scan(c, cnt):
        h16 = heads_v[pl.ds(c * 16, 16)]
        m = (h16 >= lo) & (h16 < hi)
        m2 = (h16 >= xlo) & (h16 < xhi) & (wid < 4)
        m = m | m2
        pos16 = c * 16 + lanes
        plsc.store_compressed(wl_p.at[pl.ds(cnt, 16)], pos16, mask=m)
        return cnt + plsc.all_reduce_population_count(m)[0]

    n = lax.fori_loop(0, BATCH // 16, scan, 0)
    wl_p[pl.ds(n, 16)] = jnp.full((16,), BATCH, jnp.int32)
    nch = (n + 15) // 16

    def flush():
        pltpu.async_copy(stage, hg.at[pos_v], sem2).wait()
        for s in range(8):
            pos_v[pl.ds(s * 16, 16)] = jnp.full((16,), _TRASH, jnp.int32)

    def make_process(bufs, width):
        def process(w_lo):
            def chunk(c, carry):
                p16 = wl_p[pl.ds(c * 16, 16)]
                h16 = plsc.load_gather(heads_v, [p16])
                m = (h16 >= w_lo) & (h16 < w_lo + width)
                cnt = plsc.all_reduce_population_count(m)[0]

                @pl.when(cnt > 0)
                def _():
                    plsc.store_compressed(tmp_h.at[pl.ds(0, 16)], h16, mask=m)
                    plsc.store_compressed(tmp_p.at[pl.ds(0, 16)], p16, mask=m)

                    def entry(k, c2):
                        ns = ns_ref[0]
                        kv = jnp.full((16,), k, jnp.int32)
                        h0v = plsc.load_gather(tmp_h, [kv]) - w_lo
                        p0v = plsc.load_gather(tmp_p, [kv])
                        h0 = h0v[0]
                        s = lax.rem(ns, 128)
                        sv = jnp.full((16,), s, jnp.int32)

                        def emit(bufref, coff):
                            hov = h0v - coff
                            for cc in range(4):
                                c16 = cc * 16 + lanes
                                vals = plsc.load_gather(bufref, [c16, hov])
                                plsc.store_scatter(stage, [sv, c16], vals)

                        if len(bufs) == 1:
                            emit(*bufs[0])
                        else:
                            for bufref, coff in bufs:
                                @pl.when((h0 >= coff) & (h0 < coff + _TW))
                                def _(bufref=bufref, coff=coff):
                                    emit(bufref, coff)

                        plsc.store_scatter(pos_v, [sv], p0v,
                                           mask=lanes == 0)
                        ns_ref[0] = ns + 1

                        @pl.when(lax.rem(ns + 1, 128) == 0)
                        def _():
                            flush()

                        return c2

                    lax.fori_loop(0, cnt, entry, 0)

                return carry

            lax.fori_loop(0, nch, chunk, 0)

        return process

    ns_ref[0] = 0
    proc_pair = make_process([(buf0, 0), (buf1, _TW)], 2 * _TW)
    proc_single = make_process([(buf0, 0)], _TW)
    proc_x = make_process([(buf0, 0)], 128)

    def pair(p, carry):
        base = lo + 2 * p * _TW
        cpa = pltpu.async_copy(vtu.at[:, pl.ds(base, _TW)], buf0, sem)
        cpb = pltpu.async_copy(vtu.at[:, pl.ds(base + _TW, _TW)], buf1, sem)
        cpa.wait()
        cpb.wait()
        proc_pair(base)
        return carry

    lax.fori_loop(0, _NPAIR, pair, 0)
    pltpu.sync_copy(vtu.at[:, pl.ds(lo + 2 * _NPAIR * _TW, _TW)], buf0)
    proc_single(lo + 2 * _NPAIR * _TW)

    xc = pl.multiple_of(jnp.minimum(xlo, 999808), 128)
    pltpu.sync_copy(vtu.at[:, pl.ds(xc, 128)], buf0.at[:, pl.ds(0, 128)])
    proc_x(xlo)

    @pl.when(lax.rem(ns_ref[0], 128) > 0)
    def _():
        flush()


def _make_ga():
    return pl.kernel(
        _ga_body,
        out_type=[jax.ShapeDtypeStruct((_HGR, 128), jnp.float32)],
        mesh=plsc.VectorSubcoreMesh(core_axis_name="c", subcore_axis_name="s"),
        compiler_params=pltpu.CompilerParams(needs_layout_passes=False,
                                             use_tc_tiling_on_sc=True),
        scratch_types=[
            pltpu.VMEM((BATCH + 16,), jnp.int32),
            pltpu.VMEM((BATCH + 16,), jnp.int32),
            pltpu.VMEM((64, _TW), jnp.float32),
            pltpu.VMEM((64, _TW), jnp.float32),
            pltpu.VMEM((128, 128), jnp.float32),
            pltpu.VMEM((128,), jnp.int32),
            pltpu.VMEM((32,), jnp.int32),
            pltpu.VMEM((32,), jnp.int32),
            pltpu.SMEM((1,), jnp.int32),
            pltpu.SemaphoreType.DMA,
            pltpu.SemaphoreType.DMA,
        ],
    )


def _gb_body(hg3, heads, rels, ptails, ntails, tags, trig, aux,
             possq, negsq,
             h_v, idx_p, idx_n, rel_v, head_v, trig_v, aux_v,
             p_rows, n_rows, ps_v, ns_v, sem):
    wid = lax.axis_index("s") * _NC + lax.axis_index("c")
    base = wid * _BPW

    pltpu.sync_copy(hg3.at[pl.ds(wid * 64, 64), :, pl.ds(0, DIM)], h_v)
    pltpu.sync_copy(heads.at[pl.ds(base, _BPW)], head_v)
    pltpu.sync_copy(ptails.at[wid], idx_p)
    pltpu.sync_copy(ntails.at[wid], idx_n)
    pltpu.sync_copy(rels.at[wid], rel_v)
    pltpu.sync_copy(trig, trig_v)
    pltpu.sync_copy(aux, aux_v)

    cps = []
    for k in range(_NG):
        sl = pl.ds(k * _GCH, _GCH)
        cps.append(pltpu.async_copy(tags.at[idx_p.at[k]], p_rows.at[sl], sem))
        cps.append(pltpu.async_copy(tags.at[idx_n.at[k]], n_rows.at[sl], sem))
    for cp in cps:
        cp.wait()

    lanes = lax.iota(jnp.int32, 16)

    def fixup(c, carry):
        row16 = c * 16 + lanes
        r_hi = lax.shift_right_logical(row16, 3)
        r_lo = lax.bitwise_and(row16, 7)
        heads16 = head_v[pl.ds(c * 16, 16)]

        mt = heads16 >= _AUX0
        tcnt = plsc.all_reduce_population_count(mt)[0]

        @pl.when(tcnt > 0)
        def _():
            a16 = heads16 - _AUX0
            for j in range(DIM):
                jv = jnp.full((16,), j, jnp.int32)
                vals = plsc.load_gather(aux_v, [a16, jv])
                plsc.store_scatter(h_v, [r_hi, r_lo, jv], vals, mask=mt)

        return carry

    lax.fori_loop(0, _BPW // 16, fixup, 0)

    @plsc.parallel_loop(0, _BPW // 16, unroll=4)
    def chunk(c):
        row16 = c * 16 + lanes
        r_hi = lax.shift_right_logical(row16, 3)
        r_lo = lax.bitwise_and(row16, 7)
        rel16 = rel_v[pl.ds(c * 16, 16)]

        accp = jnp.zeros((16,), jnp.float32)
        accn = jnp.zeros((16,), jnp.float32)
        for j in range(HALF):
            jv = jnp.full((16,), j, jnp.int32)
            jv2 = jnp.full((16,), j + HALF, jnp.int32)
            hre = plsc.load_gather(h_v, [r_hi, r_lo, jv])
            him = plsc.load_gather(h_v, [r_hi, r_lo, jv2])
            cj = plsc.load_gather(trig_v, [rel16, jv])
            sj = plsc.load_gather(trig_v, [rel16, jv2])
            rre = hre * cj - him * sj
            rim = hre * sj + him * cj
            pre = plsc.load_gather(p_rows, [row16, jv])
            pim = plsc.load_gather(p_rows, [row16, jv2])
            nre = plsc.load_gather(n_rows, [row16, jv])
            nim = plsc.load_gather(n_rows, [row16, jv2])
            dp0 = rre - pre
            dp1 = rim - pim
            accp = accp + dp0 * dp0 + dp1 * dp1
            dn0 = rre - nre
            dn1 = rim - nim
            accn = accn + dn0 * dn0 + dn1 * dn1
        ps_v[pl.ds(c * 16, 16)] = accp
        ns_v[pl.ds(c * 16, 16)] = accn

    pltpu.sync_copy(ps_v, possq.at[wid])
    pltpu.sync_copy(ns_v, negsq.at[wid])


def _make_gb():
    return pl.kernel(
        _gb_body,
        out_type=[
            jax.ShapeDtypeStruct((_NW, _BPW), jnp.float32),
            jax.ShapeDtypeStruct((_NW, _BPW), jnp.float32),
        ],
        mesh=plsc.VectorSubcoreMesh(core_axis_name="c", subcore_axis_name="s"),
        compiler_params=pltpu.CompilerParams(needs_layout_passes=False,
                                             use_tc_tiling_on_sc=False),
        scratch_types=[
            pltpu.VMEM((64, 8, DIM), jnp.float32),
            pltpu.VMEM((_NG, _GCH), jnp.int32),
            pltpu.VMEM((_NG, _GCH), jnp.int32),
            pltpu.VMEM((_BPW,), jnp.int32),
            pltpu.VMEM((_BPW,), jnp.int32),
            pltpu.VMEM((3, DIM), jnp.float32),
            pltpu.VMEM((64, 64), jnp.float32),
            pltpu.VMEM((_BPW, DIM), jnp.float32),
            pltpu.VMEM((_BPW, DIM), jnp.float32),
            pltpu.VMEM((_BPW,), jnp.float32),
            pltpu.VMEM((_BPW,), jnp.float32),
            pltpu.SemaphoreType.DMA,
        ],
    )


def _loss_body(ps_ref, ns_ref, out_ref):
    p = jnp.sqrt(ps_ref[...])
    n = jnp.sqrt(ns_ref[...])
    s = jnp.sum(jnp.maximum(MARGIN + p - n, 0.0))
    out_ref[...] = (s * (1.0 / BATCH)).reshape(1, 1)


_loss = pl.pallas_call(
    _loss_body,
    out_shape=jax.ShapeDtypeStruct((1, 1), jnp.float32),
)

_ga = _make_ga()
_gb = _make_gb()


def kernel(pos_heads, pos_relations, pos_tails, neg_tails, user_emb,
           hashtag_emb, rel_emb):
    heads = pos_heads.astype(jnp.int32)
    rels = pos_relations.astype(jnp.int32).reshape(_NW, _BPW)
    ptails = pos_tails.astype(jnp.int32).reshape(_NW, _NG, _GCH)
    ntails = neg_tails.astype(jnp.int32).reshape(_NW, _NG, _GCH)

    vtu = user_emb.T
    tail = lax.slice(vtu, (0, _AUX0), (64, _NU))
    cos_t, sin_t = (r := _tc_prep(rel_emb, tail))[0], r[1]
    aux = r[2]
    trig = jnp.concatenate([cos_t, sin_t], axis=-1)

    (hg,) = _ga(vtu, heads)
    hg3 = hg.reshape(_HGR // 8, 8, 128)

    possq, negsq = _gb(hg3, heads, rels, ptails, ntails, hashtag_emb,
                       trig, aux)
    return _loss(possq, negsq)[0, 0]


# speedup vs baseline: 1.2545x; 1.2545x over previous
import jax
import jax.numpy as jnp
from jax import lax
from jax.experimental import pallas as pl
from jax.experimental.pallas import tpu as pltpu
from jax.experimental.pallas import tpu_sc as plsc

BATCH = 16384
DIM = 64
HALF = 32
MARGIN = 1.0

_NC = 2
_NS = 16
_NW = _NC * _NS
_BPW = BATCH // _NW
_GCH = 128
_NG = _BPW // _GCH

_NU = 1000000
_TW = 512
_STRIPE = 31232
_NPAIR = 30
_XBASE = _NW * _STRIPE
_AUX0 = _XBASE + 4 * 128
_TRASH = BATCH
_HGR = BATCH + 128


def _tc_prep_body(rel_ref, tail_ref, cos_ref, sin_ref, aux_ref):
    r = rel_ref[...]
    cos_ref[...] = jnp.cos(r)
    sin_ref[...] = jnp.sin(r)
    aux_ref[...] = tail_ref[...].T


_tc_prep = pl.pallas_call(
    _tc_prep_body,
    out_shape=[
        jax.ShapeDtypeStruct((3, HALF), jnp.float32),
        jax.ShapeDtypeStruct((3, HALF), jnp.float32),
        jax.ShapeDtypeStruct((64, 64), jnp.float32),
    ],
)


def _ga_body(vtu, heads, hg,
             heads_v, wl_a, wl_b, buf0, buf1, stage, pos_v,
             tmp_h, tmp_p, ns_ref, sem, sem2):
    wid = lax.axis_index("s") * _NC + lax.axis_index("c")
    lo = wid * _STRIPE
    hi = lo + _STRIPE
    xlo = _XBASE + wid * 128
    xhi = xlo + 128
    lanes = lax.iota(jnp.int32, 16)

    pltpu.sync_copy(heads, heads_v.at[pl.ds(0, BATCH)])

    for s in range(7):
        pos_v[pl.ds(s * 16, 16)] = jnp.full((16,), _TRASH, jnp.int32)

    heads_v[pl.ds(BATCH, 16)] = jnp.full((16,), -1, jnp.int32)

    mid = lo + 31 * _TW

    def scan(c, carry):
        ca, cb = carry
        h16 = heads_v[pl.ds(c * 16, 16)]
        ma = (h16 >= lo) & (h16 < mid)
        mb = (h16 >= mid) & (h16 < hi)
        mb2 = (h16 >= xlo) & (h16 < xhi) & (wid < 4)
        mb = mb | mb2
        pos16 = c * 16 + lanes
        plsc.store_compressed(wl_a.at[pl.ds(ca, 16)], pos16, mask=ma)
        plsc.store_compressed(wl_b.at[pl.ds(cb, 16)], pos16, mask=mb)
        return (ca + plsc.all_reduce_population_count(ma)[0],
                cb + plsc.all_reduce_population_count(mb)[0])

    n_a, n_b = lax.fori_loop(0, BATCH // 16, scan, (0, 0))
    wl_a[pl.ds(n_a, 16)] = jnp.full((16,), BATCH, jnp.int32)
    wl_b[pl.ds(n_b, 16)] = jnp.full((16,), BATCH, jnp.int32)
    nch_a = (n_a + 15) // 16
    nch_b = (n_b + 15) // 16

    def flush():
        pltpu.async_copy(stage, hg.at[pos_v], sem2).wait()
        for s in range(7):
            pos_v[pl.ds(s * 16, 16)] = jnp.full((16,), _TRASH, jnp.int32)

    def make_process(bufref, width, wl_p, nch_f):
        def process(w_lo):
            nch = nch_f()
            def chunk(c, carry):
                p16 = wl_p[pl.ds(c * 16, 16)]
                h16 = plsc.load_gather(heads_v, [p16])
                m = (h16 >= w_lo) & (h16 < w_lo + width)
                cnt = plsc.all_reduce_population_count(m)[0]

                @pl.when(cnt > 0)
                def _():
                    plsc.store_compressed(tmp_h.at[pl.ds(0, 16)], h16, mask=m)
                    plsc.store_compressed(tmp_p.at[pl.ds(0, 16)], p16, mask=m)

                    def entry(k, c2):
                        ns = ns_ref[0]
                        h0 = tmp_h[pl.ds(k, 16)][0] - w_lo
                        p0 = tmp_p[pl.ds(k, 16)][0]
                        s = lax.rem(ns, 112)
                        h0v = jnp.full((16,), h0, jnp.int32)
                        sv = jnp.full((16,), s, jnp.int32)
                        for cc in range(4):
                            c16 = cc * 16 + lanes
                            vals = plsc.load_gather(bufref, [c16, h0v])
                            plsc.store_scatter(stage, [sv, c16], vals)
                        plsc.store_scatter(pos_v, [sv],
                                           jnp.full((16,), p0, jnp.int32),
                                           mask=lanes == 0)
                        ns_ref[0] = ns + 1

                        @pl.when(lax.rem(ns + 1, 112) == 0)
                        def _():
                            flush()

                        return c2

                    lax.fori_loop(0, cnt, entry, 0)

                return carry

            lax.fori_loop(0, nch, chunk, 0)

        return process

    ns_ref[0] = 0
    proc_a0 = make_process(buf0, _TW, wl_a, lambda: nch_a)
    proc_a1 = make_process(buf1, _TW, wl_a, lambda: nch_a)
    proc_b0 = make_process(buf0, _TW, wl_b, lambda: nch_b)
    proc_b1 = make_process(buf1, _TW, wl_b, lambda: nch_b)
    proc_x = make_process(buf0, 128, wl_b, lambda: nch_b)

    def drain(buf):
        pltpu.make_async_copy(vtu.at[:, pl.ds(0, _TW)], buf, sem).wait()

    pltpu.async_copy(vtu.at[:, pl.ds(lo, _TW)], buf0, sem)

    def make_pair(pa, pb):
        def pair(p, carry):
            base = lo + 2 * p * _TW
            pltpu.async_copy(vtu.at[:, pl.ds(base + _TW, _TW)], buf1, sem)
            drain(buf0)
            pa(base)
            nxt = pl.multiple_of(jnp.minimum(base + 2 * _TW, 999424), 128)
            pltpu.async_copy(vtu.at[:, pl.ds(nxt, _TW)], buf0, sem)
            drain(buf1)
            pb(base + _TW)
            return carry
        return pair

    lax.fori_loop(0, 15, make_pair(proc_a0, proc_a1), 0)
    make_pair(proc_a0, proc_b1)(15, 0)
    lax.fori_loop(16, _NPAIR, make_pair(proc_b0, proc_b1), 0)
    drain(buf0)
    proc_b0(lo + 2 * _NPAIR * _TW)

    xc = pl.multiple_of(jnp.minimum(xlo, 999808), 128)
    pltpu.sync_copy(vtu.at[:, pl.ds(xc, 128)], buf0.at[:, pl.ds(0, 128)])
    proc_x(xlo)

    @pl.when(lax.rem(ns_ref[0], 112) > 0)
    def _():
        flush()


def _make_ga():
    return pl.kernel(
        _ga_body,
        out_type=[jax.ShapeDtypeStruct((_HGR, 128), jnp.float32)],
        mesh=plsc.VectorSubcoreMesh(core_axis_name="c", subcore_axis_name="s"),
        compiler_params=pltpu.CompilerParams(needs_layout_passes=False,
                                             use_tc_tiling_on_sc=True),
        scratch_types=[
            pltpu.VMEM((BATCH + 16,), jnp.int32),
            pltpu.VMEM((BATCH + 16,), jnp.int32),
            pltpu.VMEM((BATCH + 16,), jnp.int32),
            pltpu.VMEM((64, _TW), jnp.float32),
            pltpu.VMEM((64, _TW), jnp.float32),
            pltpu.VMEM((112, 128), jnp.float32),
            pltpu.VMEM((112,), jnp.int32),
            pltpu.VMEM((32,), jnp.int32),
            pltpu.VMEM((32,), jnp.int32),
            pltpu.SMEM((1,), jnp.int32),
            pltpu.SemaphoreType.DMA,
            pltpu.SemaphoreType.DMA,
        ],
    )


def _gb_body(hg3, heads, rels, ptails, ntails, tags, trig, aux,
             possq, negsq,
             h_v, idx_p, idx_n, rel_v, head_v, trig_v, aux_v,
             p_rows, n_rows, ps_v, ns_v, sem):
    wid = lax.axis_index("s") * _NC + lax.axis_index("c")
    base = wid * _BPW

    pltpu.sync_copy(hg3.at[pl.ds(wid * 64, 64), :, pl.ds(0, DIM)], h_v)
    pltpu.sync_copy(heads.at[pl.ds(base, _BPW)], head_v)
    pltpu.sync_copy(ptails.at[wid], idx_p)
    pltpu.sync_copy(ntails.at[wid], idx_n)
    pltpu.sync_copy(rels.at[wid], rel_v)
    pltpu.sync_copy(trig, trig_v)
    pltpu.sync_copy(aux, aux_v)

    cps = []
    for k in range(_NG):
        sl = pl.ds(k * _GCH, _GCH)
        cps.append(pltpu.async_copy(tags.at[idx_p.at[k]], p_rows.at[sl], sem))
        cps.append(pltpu.async_copy(tags.at[idx_n.at[k]], n_rows.at[sl], sem))
    for cp in cps:
        cp.wait()

    lanes = lax.iota(jnp.int32, 16)

    def fixup(c, carry):
        row16 = c * 16 + lanes
        r_hi = lax.shift_right_logical(row16, 3)
        r_lo = lax.bitwise_and(row16, 7)
        heads16 = head_v[pl.ds(c * 16, 16)]

        mt = heads16 >= _AUX0
        tcnt = plsc.all_reduce_population_count(mt)[0]

        @pl.when(tcnt > 0)
        def _():
            a16 = heads16 - _AUX0
            for j in range(DIM):
                jv = jnp.full((16,), j, jnp.int32)
                vals = plsc.load_gather(aux_v, [a16, jv])
                plsc.store_scatter(h_v, [r_hi, r_lo, jv], vals, mask=mt)

        return carry

    lax.fori_loop(0, _BPW // 16, fixup, 0)

    @plsc.parallel_loop(0, _BPW // 16, unroll=2)
    def chunk(c):
        row16 = c * 16 + lanes
        r_hi = lax.shift_right_logical(row16, 3)
        r_lo = lax.bitwise_and(row16, 7)
        rel16 = rel_v[pl.ds(c * 16, 16)]

        accp = jnp.zeros((16,), jnp.float32)
        accn = jnp.zeros((16,), jnp.float32)
        for j in range(HALF):
            jv = jnp.full((16,), j, jnp.int32)
            jv2 = jnp.full((16,), j + HALF, jnp.int32)
            hre = plsc.load_gather(h_v, [r_hi, r_lo, jv])
            him = plsc.load_gather(h_v, [r_hi, r_lo, jv2])
            cj = plsc.load_gather(trig_v, [rel16, jv])
            sj = plsc.load_gather(trig_v, [rel16, jv2])
            rre = hre * cj - him * sj
            rim = hre * sj + him * cj
            pre = plsc.load_gather(p_rows, [row16, jv])
            pim = plsc.load_gather(p_rows, [row16, jv2])
            nre = plsc.load_gather(n_rows, [row16, jv])
            nim = plsc.load_gather(n_rows, [row16, jv2])
            dp0 = rre - pre
            dp1 = rim - pim
            accp = accp + dp0 * dp0 + dp1 * dp1
            dn0 = rre - nre
            dn1 = rim - nim
            accn = accn + dn0 * dn0 + dn1 * dn1
        ps_v[pl.ds(c * 16, 16)] = accp
        ns_v[pl.ds(c * 16, 16)] = accn

    pltpu.sync_copy(ps_v, possq.at[wid])
    pltpu.sync_copy(ns_v, negsq.at[wid])


def _make_gb():
    return pl.kernel(
        _gb_body,
        out_type=[
            jax.ShapeDtypeStruct((_NW, _BPW), jnp.float32),
            jax.ShapeDtypeStruct((_NW, _BPW), jnp.float32),
        ],
        mesh=plsc.VectorSubcoreMesh(core_axis_name="c", subcore_axis_name="s"),
        compiler_params=pltpu.CompilerParams(needs_layout_passes=False,
                                             use_tc_tiling_on_sc=False),
        scratch_types=[
            pltpu.VMEM((64, 8, DIM), jnp.float32),
            pltpu.VMEM((_NG, _GCH), jnp.int32),
            pltpu.VMEM((_NG, _GCH), jnp.int32),
            pltpu.VMEM((_BPW,), jnp.int32),
            pltpu.VMEM((_BPW,), jnp.int32),
            pltpu.VMEM((3, DIM), jnp.float32),
            pltpu.VMEM((64, 64), jnp.float32),
            pltpu.VMEM((_BPW, DIM), jnp.float32),
            pltpu.VMEM((_BPW, DIM), jnp.float32),
            pltpu.VMEM((_BPW,), jnp.float32),
            pltpu.VMEM((_BPW,), jnp.float32),
            pltpu.SemaphoreType.DMA,
        ],
    )


def _loss_body(ps_ref, ns_ref, out_ref):
    p = jnp.sqrt(ps_ref[...])
    n = jnp.sqrt(ns_ref[...])
    s = jnp.sum(jnp.maximum(MARGIN + p - n, 0.0))
    out_ref[...] = (s * (1.0 / BATCH)).reshape(1, 1)


_loss = pl.pallas_call(
    _loss_body,
    out_shape=jax.ShapeDtypeStruct((1, 1), jnp.float32),
)

_ga = _make_ga()
_gb = _make_gb()


def kernel(pos_heads, pos_relations, pos_tails, neg_tails, user_emb,
           hashtag_emb, rel_emb):
    heads = pos_heads.astype(jnp.int32)
    rels = pos_relations.astype(jnp.int32).reshape(_NW, _BPW)
    ptails = pos_tails.astype(jnp.int32).reshape(_NW, _NG, _GCH)
    ntails = neg_tails.astype(jnp.int32).reshape(_NW, _NG, _GCH)

    vtu = user_emb.T
    tail = lax.slice(vtu, (0, _AUX0), (64, _NU))
    cos_t, sin_t = (r := _tc_prep(rel_emb, tail))[0], r[1]
    aux = r[2]
    trig = jnp.concatenate([cos_t, sin_t], axis=-1)

    (hg,) = _ga(vtu, heads)
    hg3 = hg.reshape(_HGR // 8, 8, 128)

    possq, negsq = _gb(hg3, heads, rels, ptails, ntails, hashtag_emb,
                       trig, aux)
    return _loss(possq, negsq)[0, 0]
